# baseline (speedup 1.0000x reference)
import jax
import jax.numpy as jnp
from jax import lax
from jax.experimental import pallas as pl
from jax.experimental.pallas import tpu as pltpu

_MARGIN_P = 0.5
_MARGIN_N = 1.5
_EPS = 1e-09

_N = 4096
_D = 128
_CH = 512
_NC = _N // _CH
_GPC = _CH // 128


def _body(o1_hbm, o2_hbm, tgt_ref, out_ref, nv_s, b1, b2, sems):
    def copies(c):
        rows = pl.ds(c * _CH, _CH)
        return (
            pltpu.make_async_copy(o1_hbm.at[rows, :], b1.at[c], sems.at[0, c]),
            pltpu.make_async_copy(o2_hbm.at[rows, :], b2.at[c], sems.at[1, c]),
        )

    for c in range(_NC):
        for cp in copies(c):
            cp.start()

    ones_c = jnp.ones((_D, 1), jnp.float32)
    tgt = tgt_ref[0]
    accp = jnp.zeros((1, 128), jnp.float32)
    accm = jnp.full((1, 128), -1.0, jnp.float32)
    for c in range(_NC):
        for cp in copies(c):
            cp.wait()
        for j in range(_GPC):
            g = c * _GPC + j
            diff = (b2[c, pl.ds(j * 128, 128), :]
                    - b1[c, pl.ds(j * 128, 128), :])
            sq = diff * diff
            d = lax.dot_general(
                ones_c, sq, (((0,), (1,)), ((), ())),
                preferred_element_type=jnp.float32,
            )
            s = jnp.sqrt(d + _EPS)
            loss_p = 0.5 * jnp.maximum(s - _MARGIN_P, 0.0) ** 2
            loss_n = 0.5 * jnp.maximum(_MARGIN_N - s, 0.0) ** 2
            tgt_g = tgt[:, g * 128:(g + 1) * 128]
            mask = tgt_g != 0
            nv_j = jnp.where(mask, jnp.float32(-1.0), loss_n)
            nv_s[pl.ds(g, 1), :] = nv_j
            accm = jnp.maximum(accm, nv_j)
            accp = accp + jnp.where(mask, loss_p, 0.0)

    num_pos = jnp.sum(tgt)
    n_neg = _N - num_pos
    k = jnp.minimum(jnp.maximum(1, num_pos), n_neg)
    maxv = jnp.max(accm)

    def _select():
        nv = nv_s[...]
        hi0 = lax.bitcast_convert_type(maxv, jnp.int32) + 1
        lo0 = jnp.int32(0)

        def w_cond(st):
            lo, hi = st
            return (hi - lo) > 1

        def w_body(st):
            lo, hi = st
            mid = lo + ((hi - lo) >> 1)
            t = lax.bitcast_convert_type(mid, jnp.float32)
            ge = jnp.sum((nv >= t).astype(jnp.int32))
            take = ge >= k
            return jnp.where(take, mid, lo), jnp.where(take, hi, mid)

        t_bits, _ = lax.while_loop(w_cond, w_body, (lo0, hi0))
        t = lax.bitcast_convert_type(t_bits, jnp.float32)
        gt = nv > t
        sum_gt = jnp.sum(jnp.where(gt, nv, 0.0))
        cnt_gt = jnp.sum(gt.astype(jnp.int32))
        return sum_gt + t * (k - cnt_gt).astype(jnp.float32)

    sum_n = lax.cond(maxv > 0.0, _select, lambda: jnp.float32(0.0))
    sum_p = jnp.sum(accp)
    total = (sum_p + sum_n) / (num_pos + k).astype(jnp.float32)
    out_ref[...] = jnp.full((1, 1), total, jnp.float32)


@jax.jit
def _run(output1, output2, target):
    tgt3d = target.reshape(1, 1, _N)
    out = pl.pallas_call(
        _body,
        in_specs=[
            pl.BlockSpec(memory_space=pl.ANY),
            pl.BlockSpec(memory_space=pl.ANY),
            pl.BlockSpec((1, 1, _N), lambda: (0, 0, 0)),
        ],
        out_specs=pl.BlockSpec((1, 1), lambda: (0, 0)),
        out_shape=jax.ShapeDtypeStruct((1, 1), jnp.float32),
        scratch_shapes=[
            pltpu.VMEM((_N // 128, 128), jnp.float32),
            pltpu.VMEM((_NC, _CH, _D), jnp.float32),
            pltpu.VMEM((_NC, _CH, _D), jnp.float32),
            pltpu.SemaphoreType.DMA((2, _NC)),
        ],
    )(output1, output2, tgt3d)
    return out[0, 0]


def kernel(output1, output2, target):
    return _run(output1, output2, target)


# speedup vs baseline: 1.3254x; 1.3254x over previous
import jax
import jax.numpy as jnp
from jax import lax
from jax.experimental import pallas as pl
from jax.experimental.pallas import tpu as pltpu

_MARGIN_P = 0.5
_MARGIN_N = 1.5
_EPS = 1e-09

_N = 4096
_D = 128
_SUB = 16
_ROWS = 128 * _SUB
_GRID = _N // _ROWS


def _body(o1_ref, o2_ref, tgt_ref, out_ref, nv_s, accp_s, accnp_s, accm_s):
    i = pl.program_id(0)

    @pl.when(i == 0)
    def _init():
        accp_s[...] = jnp.zeros((1, 128), jnp.float32)
        accnp_s[...] = jnp.zeros((1, 128), jnp.int32)
        accm_s[...] = jnp.full((1, 128), -1.0, jnp.float32)

    ones_c = jnp.ones((_D, 1), jnp.float32)
    tgt = tgt_ref[0]
    accp = accp_s[...]
    accnp = accnp_s[...]
    accm = accm_s[...]
    for j in range(_SUB):
        diff = o2_ref[pl.ds(j * 128, 128), :] - o1_ref[pl.ds(j * 128, 128), :]
        sq = diff * diff
        d = lax.dot_general(
            ones_c, sq, (((0,), (1,)), ((), ())),
            preferred_element_type=jnp.float32,
        )
        s = jnp.sqrt(d + _EPS)
        loss_p = 0.5 * jnp.maximum(s - _MARGIN_P, 0.0) ** 2
        loss_n = 0.5 * jnp.maximum(_MARGIN_N - s, 0.0) ** 2
        tgt_j = tgt[:, j * 128:(j + 1) * 128]
        mask = tgt_j != 0
        nv_j = jnp.where(mask, jnp.float32(-1.0), loss_n)
        nv_s[pl.ds(i * _SUB + j, 1), :] = nv_j
        accm = jnp.maximum(accm, nv_j)
        accp = accp + jnp.where(mask, loss_p, 0.0)
        accnp = accnp + tgt_j
    accp_s[...] = accp
    accnp_s[...] = accnp
    accm_s[...] = accm

    @pl.when(i == _GRID - 1)
    def _finish():
        num_pos = jnp.sum(accnp_s[...])
        n_neg = _N - num_pos
        k = jnp.minimum(jnp.maximum(1, num_pos), n_neg)
        maxv = jnp.max(accm_s[...])

        def _select():
            nv = nv_s[...]
            hi0 = lax.bitcast_convert_type(maxv, jnp.int32) + 1
            lo0 = jnp.int32(0)

            def w_cond(st):
                lo, hi = st
                return (hi - lo) > 1

            def w_body(st):
                lo, hi = st
                mid = lo + ((hi - lo) >> 1)
                t = lax.bitcast_convert_type(mid, jnp.float32)
                ge = jnp.sum((nv >= t).astype(jnp.int32))
                take = ge >= k
                return jnp.where(take, mid, lo), jnp.where(take, hi, mid)

            t_bits, _ = lax.while_loop(w_cond, w_body, (lo0, hi0))
            t = lax.bitcast_convert_type(t_bits, jnp.float32)

            gt = nv > t
            sum_gt = jnp.sum(jnp.where(gt, nv, 0.0))
            cnt_gt = jnp.sum(gt.astype(jnp.int32))
            return sum_gt + t * (k - cnt_gt).astype(jnp.float32)

        def _zero():
            return jnp.float32(0.0)

        sum_n = lax.cond(maxv > 0.0, _select, _zero)

        sum_p = jnp.sum(accp_s[...])
        total = (sum_p + sum_n) / (num_pos + k).astype(jnp.float32)
        out_ref[...] = jnp.full((1, 1), total, jnp.float32)


@jax.jit
def _run(output1, output2, target):
    tgt3d = target.reshape(_GRID, 1, _SUB * 128)
    out = pl.pallas_call(
        _body,
        grid=(_GRID,),
        in_specs=[
            pl.BlockSpec((_ROWS, _D), lambda i: (i, 0)),
            pl.BlockSpec((_ROWS, _D), lambda i: (i, 0)),
            pl.BlockSpec((1, 1, _SUB * 128), lambda i: (i, 0, 0)),
        ],
        out_specs=pl.BlockSpec((1, 1), lambda i: (0, 0)),
        out_shape=jax.ShapeDtypeStruct((1, 1), jnp.float32),
        scratch_shapes=[
            pltpu.VMEM((_N // 128, 128), jnp.float32),
            pltpu.VMEM((1, 128), jnp.float32),
            pltpu.VMEM((1, 128), jnp.int32),
            pltpu.VMEM((1, 128), jnp.float32),
        ],
    )(output1, output2, tgt3d)
    return out[0, 0]


def kernel(output1, output2, target):
    return _run(output1, output2, target)
